# baseline (speedup 1.0000x reference)
import functools

import jax
import jax.numpy as jnp
from jax import lax
from jax.experimental import pallas as pl
from jax.experimental.pallas import tpu as pltpu
from jax.experimental.pallas import tpu_sc as plsc

VOCAB = 100000
EMB = 16
CTX = 5
HID = 64
B = 1024

_info = plsc.get_sparse_core_info()
_NC, _NS = _info.num_cores, _info.num_subcores
_NW = _NC * _NS
_NIDX = B * CTX
_B_PER_W = _NIDX // _NW


def _gather_body(table_hbm, idx_hbm, out_hbm, idx_v, rows_v, sem):
    wid = lax.axis_index("s") * _NC + lax.axis_index("c")
    base = wid * _B_PER_W
    pltpu.sync_copy(idx_hbm.at[pl.ds(base, _B_PER_W)], idx_v)
    pltpu.async_copy(table_hbm.at[idx_v], rows_v, sem).wait()
    pltpu.sync_copy(rows_v, out_hbm.at[pl.ds(base, _B_PER_W)])


_sc_gather = functools.partial(
    pl.kernel,
    mesh=plsc.VectorSubcoreMesh(core_axis_name="c", subcore_axis_name="s"),
    out_type=jax.ShapeDtypeStruct((_NIDX, EMB), jnp.float32),
    scratch_types=[
        pltpu.VMEM((_B_PER_W,), jnp.int32),
        pltpu.VMEM((_B_PER_W, EMB), jnp.float32),
        pltpu.SemaphoreType.DMA,
    ],
    compiler_params=pltpu.CompilerParams(use_tc_tiling_on_sc=False),
)(_gather_body)


V_TILE = 1024


def _mlp_body(embeds_ref, W1_ref, b1_ref, W2_ref, b2_ref, out_ref, hidden):
    @pl.when(pl.program_id(0) == 0)
    def _():
        pre = jnp.dot(embeds_ref[...], W1_ref[...],
                      preferred_element_type=jnp.float32)
        hidden[...] = jnp.maximum(pre + b1_ref[...], 0.0)

    out_ref[...] = jnp.dot(hidden[...], W2_ref[...],
                           preferred_element_type=jnp.float32) + b2_ref[...]


def _mlp(embeds, W1, b1, W2, b2):
    nv = pl.cdiv(VOCAB, V_TILE)
    return pl.pallas_call(
        _mlp_body,
        grid=(nv,),
        in_specs=[
            pl.BlockSpec((B, CTX * EMB), lambda j: (0, 0)),
            pl.BlockSpec((CTX * EMB, HID), lambda j: (0, 0)),
            pl.BlockSpec((1, HID), lambda j: (0, 0)),
            pl.BlockSpec((HID, V_TILE), lambda j: (0, j)),
            pl.BlockSpec((1, V_TILE), lambda j: (0, j)),
        ],
        out_specs=pl.BlockSpec((B, V_TILE), lambda j: (0, j)),
        out_shape=jax.ShapeDtypeStruct((B, VOCAB), jnp.float32),
        scratch_shapes=[pltpu.VMEM((B, HID), jnp.float32)],
    )(embeds, W1, b1, W2, b2)


def kernel(inputs, emb, W1, b1, W2, b2):
    rows = _sc_gather(emb, inputs.reshape(-1))
    embeds = rows.reshape(B, CTX * EMB)
    return _mlp(embeds, W1, b1.reshape(1, HID), W2, b2.reshape(1, VOCAB))


# speedup vs baseline: 1.0419x; 1.0419x over previous
import functools

import jax
import jax.numpy as jnp
from jax import lax
from jax.experimental import pallas as pl
from jax.experimental.pallas import tpu as pltpu
from jax.experimental.pallas import tpu_sc as plsc

VOCAB = 100000
EMB = 16
CTX = 5
HID = 64
B = 1024

_info = plsc.get_sparse_core_info()
_NC, _NS = _info.num_cores, _info.num_subcores
_NW = _NC * _NS
_NIDX = B * CTX
_B_PER_W = _NIDX // _NW


def _gather_body(table_hbm, idx_hbm, out_hbm, idx_v, rows_v, sem):
    wid = lax.axis_index("s") * _NC + lax.axis_index("c")
    base = wid * _B_PER_W
    pltpu.sync_copy(idx_hbm.at[pl.ds(base, _B_PER_W)], idx_v)
    pltpu.async_copy(table_hbm.at[idx_v], rows_v, sem).wait()
    pltpu.sync_copy(rows_v, out_hbm.at[pl.ds(base, _B_PER_W)])


_sc_gather = functools.partial(
    pl.kernel,
    mesh=plsc.VectorSubcoreMesh(core_axis_name="c", subcore_axis_name="s"),
    out_type=jax.ShapeDtypeStruct((_NIDX, EMB), jnp.float32),
    scratch_types=[
        pltpu.VMEM((_B_PER_W,), jnp.int32),
        pltpu.VMEM((_B_PER_W, EMB), jnp.float32),
        pltpu.SemaphoreType.DMA,
    ],
    compiler_params=pltpu.CompilerParams(use_tc_tiling_on_sc=False),
)(_gather_body)


V_TILE = 2048


def _mlp_body(embeds_ref, W1_ref, b1_ref, W2_ref, b2_ref, out_ref, hidden):
    @pl.when(pl.program_id(0) == 0)
    def _():
        pre = jnp.dot(embeds_ref[...], W1_ref[...],
                      preferred_element_type=jnp.float32)
        hidden[...] = jnp.maximum(pre + b1_ref[...], 0.0)

    out_ref[...] = jnp.dot(hidden[...], W2_ref[...],
                           preferred_element_type=jnp.float32) + b2_ref[...]


def _mlp(embeds, W1, b1, W2, b2):
    nv = pl.cdiv(VOCAB, V_TILE)
    return pl.pallas_call(
        _mlp_body,
        grid=(nv,),
        in_specs=[
            pl.BlockSpec((B, CTX * EMB), lambda j: (0, 0)),
            pl.BlockSpec((CTX * EMB, HID), lambda j: (0, 0)),
            pl.BlockSpec((1, HID), lambda j: (0, 0)),
            pl.BlockSpec((HID, V_TILE), lambda j: (0, j)),
            pl.BlockSpec((1, V_TILE), lambda j: (0, j)),
        ],
        out_specs=pl.BlockSpec((B, V_TILE), lambda j: (0, j)),
        out_shape=jax.ShapeDtypeStruct((B, VOCAB), jnp.float32),
        scratch_shapes=[pltpu.VMEM((B, HID), jnp.float32)],
    )(embeds, W1, b1, W2, b2)


def kernel(inputs, emb, W1, b1, W2, b2):
    rows = _sc_gather(emb, inputs.reshape(-1))
    embeds = rows.reshape(B, CTX * EMB)
    return _mlp(embeds, W1, b1.reshape(1, HID), W2, b2.reshape(1, VOCAB))
